# baseline (device time: 33983 ns/iter reference)
import jax
import jax.numpy as jnp
from jax import lax
from jax.experimental import pallas as pl
from jax.experimental.pallas import tpu as pltpu

N_DEV = 4


def kernel(x, Win0, Wout0, Win1, Wout1, Win2, Wout2):
    B, D = x.shape
    H = Win0.shape[1]
    rows = B // N_DEV

    HH = H // 2
    S = 5

    def body(x_ref, win0, wout0, win1, wout1, win2, wout2, out_ref,
             wbuf, ar_buf, rs_stage, rs_buf,
             wsems,
             ar_send_sems, ar_recv_sems, rs_send_sems, rs_recv_sems):
        my = lax.axis_index("i")
        wins = [win0, win1, win2]
        wouts = [wout0, wout1, wout2]

        srcs = []
        for r in range(3):
            for j in range(2):
                srcs.append(wins[r].at[:, pl.ds(j * HH, HH)])
                srcs.append(wouts[r].at[pl.ds(j * HH, HH), :])
        n_chunks = len(srcs)
        dmas = [
            pltpu.make_async_copy(srcs[i], wbuf.at[i % S], wsems.at[i % S])
            for i in range(n_chunks)
        ]

        for i in range(S):
            dmas[i].start()

        barrier_sem = pltpu.get_barrier_semaphore()
        for k in range(1, N_DEV):
            pl.semaphore_signal(
                barrier_sem, inc=1,
                device_id=((my + k) % N_DEV,),
                device_id_type=pl.DeviceIdType.MESH,
            )
        pl.semaphore_wait(barrier_sem, N_DEV - 1)

        xb = x_ref[...].astype(jnp.bfloat16)

        def use(i):
            dmas[i].wait()
            return wbuf[i % S].astype(jnp.bfloat16)

        def consumed(i):
            if i + S < n_chunks:
                dmas[i + S].start()

        for r in range(3):
            base = 4 * r
            h0 = jnp.dot(xb, use(base), preferred_element_type=jnp.float32)
            h0 = jnp.maximum(h0, 0.0).astype(jnp.bfloat16)
            consumed(base)
            p = jnp.dot(h0, use(base + 1), preferred_element_type=jnp.float32)
            consumed(base + 1)
            h1 = jnp.dot(xb, use(base + 2), preferred_element_type=jnp.float32)
            h1 = jnp.maximum(h1, 0.0).astype(jnp.bfloat16)
            consumed(base + 2)
            p = p + jnp.dot(h1, use(base + 3),
                            preferred_element_type=jnp.float32)
            consumed(base + 3)

            if r < 2:
                ar_buf[r, 0] = p.astype(jnp.bfloat16)
                rdmas = []
                for k in range(1, N_DEV):
                    rdma = pltpu.make_async_remote_copy(
                        src_ref=ar_buf.at[r, 0],
                        dst_ref=ar_buf.at[r, k],
                        send_sem=ar_send_sems.at[r, k],
                        recv_sem=ar_recv_sems.at[r, k],
                        device_id=((my + k) % N_DEV,),
                        device_id_type=pl.DeviceIdType.MESH,
                    )
                    rdma.start()
                    rdmas.append(rdma)
                for rdma in rdmas:
                    rdma.wait_recv()
                total = p
                for k in range(1, N_DEV):
                    total = total + ar_buf[r, k].astype(jnp.float32)
                for rdma in rdmas:
                    rdma.wait_send()
                xb = total.astype(jnp.bfloat16)
            else:
                rs_stage[...] = p.astype(jnp.bfloat16)
                rs_rdmas = []
                for k in range(1, N_DEV):
                    dest = (my + k) % N_DEV
                    rdma = pltpu.make_async_remote_copy(
                        src_ref=rs_stage.at[pl.ds(dest * rows, rows)],
                        dst_ref=rs_buf.at[k],
                        send_sem=rs_send_sems.at[k],
                        recv_sem=rs_recv_sems.at[k],
                        device_id=(dest,),
                        device_id_type=pl.DeviceIdType.MESH,
                    )
                    rdma.start()
                    rs_rdmas.append(rdma)
                for rdma in rs_rdmas:
                    rdma.wait_recv()
                total = rs_stage[pl.ds(my * rows, rows)].astype(jnp.float32)
                for k in range(1, N_DEV):
                    total = total + rs_buf[k].astype(jnp.float32)
                for rdma in rs_rdmas:
                    rdma.wait_send()
                out_ref[...] = total

    return pl.pallas_call(
        body,
        out_shape=jax.ShapeDtypeStruct((rows, D), jnp.float32),
        in_specs=[pl.BlockSpec(memory_space=pltpu.VMEM)]
        + [pl.BlockSpec(memory_space=pl.ANY)] * 6,
        out_specs=pl.BlockSpec(memory_space=pltpu.VMEM),
        scratch_shapes=[
            pltpu.VMEM((S, HH, D), jnp.float32),
            pltpu.VMEM((2, N_DEV, B, D), jnp.bfloat16),
            pltpu.VMEM((B, D), jnp.bfloat16),
            pltpu.VMEM((N_DEV, rows, D), jnp.bfloat16),
            pltpu.SemaphoreType.DMA((S,)),
            pltpu.SemaphoreType.DMA((2, N_DEV)),
            pltpu.SemaphoreType.DMA((2, N_DEV)),
            pltpu.SemaphoreType.DMA((N_DEV,)),
            pltpu.SemaphoreType.DMA((N_DEV,)),
        ],
        compiler_params=pltpu.CompilerParams(collective_id=0),
    )(x, Win0, Wout0, Win1, Wout1, Win2, Wout2)
